# baseline (speedup 1.0000x reference)
import functools

import jax
import jax.numpy as jnp
from jax.experimental import pallas as pl
from jax.experimental.pallas import tpu as pltpu
from jax.experimental.pallas import tpu_sc as plsc


def _main(x_ref, w_ref, b_ref, out_ref, acc_ref, *, s_len, n_chunks):
    k = pl.program_id(0)

    @pl.when(k == 0)
    def _init():
        acc_ref[...] = jnp.zeros_like(acc_ref)

    state_chunk = jnp.sum(x_ref[...], axis=1) * (1.0 / s_len)
    acc_ref[...] += jax.lax.dot_general(
        state_chunk, w_ref[...],
        dimension_numbers=(((1,), (1,)), ((), ())),
        preferred_element_type=jnp.float32)

    @pl.when(k == n_chunks - 1)
    def _finish():
        out_ref[...] = acc_ref[...] + b_ref[...]


def _tail(proj_ref, emb_ref, onehot_ref, out_ref):
    logits = jax.lax.dot_general(
        proj_ref[...], emb_ref[...],
        dimension_numbers=(((1,), (1,)), ((), ())),
        preferred_element_type=jnp.float32)
    m = jnp.max(logits, axis=1, keepdims=True)
    lse = jnp.log(jnp.sum(jnp.exp(logits - m), axis=1, keepdims=True)) + m
    logp = logits - lse
    picked = jnp.sum(logp * onehot_ref[...], axis=1, keepdims=True)
    out_ref[...] = jnp.sum(picked, axis=0, keepdims=True) * (-0.001 / logits.shape[0])


def _sc_gather(representations, indices128):
    E, H = representations.shape
    mesh = plsc.VectorSubcoreMesh(core_axis_name="core",
                                  subcore_axis_name="subcore")
    n_units = mesh.num_cores * mesh.num_subcores
    gw = max(1, E // n_units)

    @pl.kernel(out_type=jax.ShapeDtypeStruct((E, H), representations.dtype),
               mesh=mesh,
               scratch_types=[pltpu.VMEM((1, 128), jnp.int32),
                              pltpu.VMEM((gw, H), representations.dtype)])
    def gather_kernel(rep_hbm, idx_hbm, o_hbm, idx_vmem, buf):
        core = jax.lax.axis_index("core")
        sub = jax.lax.axis_index("subcore")
        start = (core * mesh.num_subcores + sub) * gw

        @pl.when(start < E)
        def _():
            pltpu.sync_copy(idx_hbm, idx_vmem)
            pltpu.sync_copy(rep_hbm.at[idx_vmem.at[0, pl.ds(start, gw)]], buf)
            pltpu.sync_copy(buf, o_hbm.at[pl.ds(start, gw)])

    return gather_kernel(representations, indices128)


def kernel(hidden_states, representations, W, b, current_indices,
           current_expert_idx, current_depth):
    B, S, H = hidden_states.shape
    E = representations.shape[0]
    C = 256
    n = H // C

    idx128 = jnp.zeros((1, 128), jnp.int32).at[0, :E].set(
        current_indices.astype(jnp.int32))
    emb = _sc_gather(representations, idx128)
    onehot = (jax.lax.iota(jnp.int32, E)[None, :]
              == jnp.asarray(current_expert_idx, jnp.int32)).astype(jnp.float32)
    b2 = b.reshape(1, H)

    proj = pl.pallas_call(
        functools.partial(_main, s_len=S, n_chunks=n),
        grid=(n,),
        in_specs=[
            pl.BlockSpec((B, S, C), lambda k: (0, 0, k)),
            pl.BlockSpec((H, C), lambda k: (0, k)),
            pl.BlockSpec((1, H), lambda k: (0, 0)),
        ],
        out_specs=pl.BlockSpec((B, H), lambda k: (0, 0)),
        out_shape=jax.ShapeDtypeStruct((B, H), jnp.float32),
        scratch_shapes=[pltpu.VMEM((B, H), jnp.float32)],
    )(hidden_states, W, b2)

    out = pl.pallas_call(
        _tail,
        out_shape=jax.ShapeDtypeStruct((1, 1), jnp.float32),
    )(proj, emb, onehot)
    return out[0, 0]


# speedup vs baseline: 1.0184x; 1.0184x over previous
import functools

import jax
import jax.numpy as jnp
from jax.experimental import pallas as pl
from jax.experimental.pallas import tpu as pltpu
from jax.experimental.pallas import tpu_sc as plsc


def _main(x_ref, w_ref, b_ref, out_ref, acc_ref, *, s_len, n_chunks):
    k = pl.program_id(0)

    @pl.when(k == 0)
    def _init():
        acc_ref[...] = jnp.zeros_like(acc_ref)

    state_chunk = jnp.sum(x_ref[...], axis=1) * (1.0 / s_len)
    acc_ref[...] += jax.lax.dot_general(
        state_chunk, w_ref[...],
        dimension_numbers=(((1,), (1,)), ((), ())),
        preferred_element_type=jnp.float32)

    @pl.when(k == n_chunks - 1)
    def _finish():
        out_ref[...] = acc_ref[...] + b_ref[...]


def _tail(proj_ref, emb_ref, onehot_ref, out_ref):
    logits = jax.lax.dot_general(
        proj_ref[...], emb_ref[...],
        dimension_numbers=(((1,), (1,)), ((), ())),
        preferred_element_type=jnp.float32)
    m = jnp.max(logits, axis=1, keepdims=True)
    lse = jnp.log(jnp.sum(jnp.exp(logits - m), axis=1, keepdims=True)) + m
    logp = logits - lse
    picked = jnp.sum(logp * onehot_ref[...], axis=1, keepdims=True)
    out_ref[...] = jnp.sum(picked, axis=0, keepdims=True) * (-0.001 / logits.shape[0])


def _sc_gather(representations, indices128):
    E, H = representations.shape
    n_cores = 2
    per_core = E // n_cores
    mesh = plsc.ScalarSubcoreMesh(axis_name="core", num_cores=n_cores)

    @pl.kernel(out_type=jax.ShapeDtypeStruct((E, H), representations.dtype),
               mesh=mesh,
               scratch_types=[pltpu.SMEM((128,), jnp.int32),
                              pltpu.SemaphoreType.DMA])
    def gather_kernel(rep_hbm, idx_hbm, o_hbm, idx_smem, sem):
        core = jax.lax.axis_index("core")
        pltpu.async_copy(idx_hbm, idx_smem, sem).wait()

        @pl.loop(0, per_core)
        def _issue(i):
            r = core * per_core + i
            pltpu.async_copy(rep_hbm.at[idx_smem[r]], o_hbm.at[r], sem)

        @pl.loop(0, per_core)
        def _wait(i):
            r = core * per_core + i
            pltpu.make_async_copy(rep_hbm.at[idx_smem[r]], o_hbm.at[r],
                                  sem).wait()

    return gather_kernel(representations, indices128.reshape(128))


def kernel(hidden_states, representations, W, b, current_indices,
           current_expert_idx, current_depth):
    B, S, H = hidden_states.shape
    E = representations.shape[0]
    C = 256
    n = H // C

    idx128 = jnp.zeros((1, 128), jnp.int32).at[0, :E].set(
        current_indices.astype(jnp.int32))
    emb = _sc_gather(representations, idx128)
    onehot = (jax.lax.iota(jnp.int32, E)[None, :]
              == jnp.asarray(current_expert_idx, jnp.int32)).astype(jnp.float32)
    b2 = b.reshape(1, H)

    proj = pl.pallas_call(
        functools.partial(_main, s_len=S, n_chunks=n),
        grid=(n,),
        in_specs=[
            pl.BlockSpec((B, S, C), lambda k: (0, 0, k)),
            pl.BlockSpec((H, C), lambda k: (0, k)),
            pl.BlockSpec((1, H), lambda k: (0, 0)),
        ],
        out_specs=pl.BlockSpec((B, H), lambda k: (0, 0)),
        out_shape=jax.ShapeDtypeStruct((B, H), jnp.float32),
        scratch_shapes=[pltpu.VMEM((B, H), jnp.float32)],
    )(hidden_states, W, b2)

    out = pl.pallas_call(
        _tail,
        out_shape=jax.ShapeDtypeStruct((1, 1), jnp.float32),
    )(proj, emb, onehot)
    return out[0, 0]


# speedup vs baseline: 1.2391x; 1.2167x over previous
import functools

import jax
import jax.numpy as jnp
from jax.experimental import pallas as pl
from jax.experimental.pallas import tpu as pltpu
from jax.experimental.pallas import tpu_sc as plsc


def _main(x_ref, w_ref, b_ref, out_ref, acc_ref, *, s_len, n_chunks):
    k = pl.program_id(0)

    @pl.when(k == 0)
    def _init():
        acc_ref[...] = jnp.zeros_like(acc_ref)

    state_chunk = jnp.sum(x_ref[...], axis=1) * (1.0 / s_len)
    acc_ref[...] += jax.lax.dot_general(
        state_chunk, w_ref[...],
        dimension_numbers=(((1,), (1,)), ((), ())),
        preferred_element_type=jnp.float32)

    @pl.when(k == n_chunks - 1)
    def _finish():
        out_ref[...] = acc_ref[...] + b_ref[...]


def _tail(proj_ref, emb_ref, onehot_ref, out_ref):
    logits = jax.lax.dot_general(
        proj_ref[...], emb_ref[...],
        dimension_numbers=(((1,), (1,)), ((), ())),
        preferred_element_type=jnp.float32)
    m = jnp.max(logits, axis=1, keepdims=True)
    lse = jnp.log(jnp.sum(jnp.exp(logits - m), axis=1, keepdims=True)) + m
    logp = logits - lse
    picked = jnp.sum(logp * onehot_ref[...], axis=1, keepdims=True)
    out_ref[...] = jnp.sum(picked, axis=0, keepdims=True) * (-0.001 / logits.shape[0])


def _sc_gather(representations, indices128):
    E, H = representations.shape
    n_cores = 2
    per_core = E // n_cores
    mesh = plsc.ScalarSubcoreMesh(axis_name="core", num_cores=n_cores)

    @pl.kernel(out_type=jax.ShapeDtypeStruct((E, H), representations.dtype),
               mesh=mesh,
               scratch_types=[pltpu.SMEM((128,), jnp.int32),
                              pltpu.SemaphoreType.DMA])
    def gather_kernel(rep_hbm, idx_hbm, o_hbm, idx_smem, sem):
        core = jax.lax.axis_index("core")
        pltpu.async_copy(idx_hbm, idx_smem, sem).wait()

        @pl.loop(0, per_core)
        def _issue(i):
            r = core * per_core + i
            pltpu.async_copy(rep_hbm.at[idx_smem[r]], o_hbm.at[r], sem)

        @pl.loop(0, per_core)
        def _wait(i):
            r = core * per_core + i
            pltpu.make_async_copy(rep_hbm.at[idx_smem[r]], o_hbm.at[r],
                                  sem).wait()

    return gather_kernel(representations, indices128.reshape(128))


def kernel(hidden_states, representations, W, b, current_indices,
           current_expert_idx, current_depth):
    B, S, H = hidden_states.shape
    E = representations.shape[0]
    C = 256
    n = H // C

    emb = jnp.take(representations, current_indices, axis=0)
    onehot = (jax.lax.iota(jnp.int32, E)[None, :]
              == jnp.asarray(current_expert_idx, jnp.int32)).astype(jnp.float32)
    b2 = b.reshape(1, H)

    proj = pl.pallas_call(
        functools.partial(_main, s_len=S, n_chunks=n),
        grid=(n,),
        in_specs=[
            pl.BlockSpec((B, S, C), lambda k: (0, 0, k)),
            pl.BlockSpec((H, C), lambda k: (0, k)),
            pl.BlockSpec((1, H), lambda k: (0, 0)),
        ],
        out_specs=pl.BlockSpec((B, H), lambda k: (0, 0)),
        out_shape=jax.ShapeDtypeStruct((B, H), jnp.float32),
        scratch_shapes=[pltpu.VMEM((B, H), jnp.float32)],
    )(hidden_states, W, b2)

    out = pl.pallas_call(
        _tail,
        out_shape=jax.ShapeDtypeStruct((1, 1), jnp.float32),
    )(proj, emb, onehot)
    return out[0, 0]


# speedup vs baseline: 1.2540x; 1.0120x over previous
import functools

import jax
import jax.numpy as jnp
from jax.experimental import pallas as pl
from jax.experimental.pallas import tpu as pltpu


def _fused(x_ref, w_ref, emb_ref, b_ref, onehot_ref, out_ref, acc_ref, *,
           s_len, n_chunks):
    k = pl.program_id(0)

    @pl.when(k == 0)
    def _init():
        acc_ref[...] = jnp.zeros_like(acc_ref)

    state_chunk = jnp.sum(x_ref[...], axis=1) * (1.0 / s_len)
    g_chunk = jax.lax.dot_general(
        emb_ref[...], w_ref[...],
        dimension_numbers=(((1,), (0,)), ((), ())),
        preferred_element_type=jnp.float32)
    acc_ref[...] += jax.lax.dot_general(
        state_chunk, g_chunk,
        dimension_numbers=(((1,), (1,)), ((), ())),
        preferred_element_type=jnp.float32)

    @pl.when(k == n_chunks - 1)
    def _finish():
        bias_logit = jax.lax.dot_general(
            b_ref[...], emb_ref[...],
            dimension_numbers=(((1,), (1,)), ((), ())),
            preferred_element_type=jnp.float32)
        logits = acc_ref[...] + bias_logit
        m = jnp.max(logits, axis=1, keepdims=True)
        lse = jnp.log(jnp.sum(jnp.exp(logits - m), axis=1, keepdims=True)) + m
        logp = logits - lse
        picked = jnp.sum(logp * onehot_ref[...], axis=1, keepdims=True)
        out_ref[...] = jnp.sum(picked, axis=0, keepdims=True) * (
            -0.001 / logits.shape[0])


def kernel(hidden_states, representations, W, b, current_indices,
           current_expert_idx, current_depth):
    B, S, H = hidden_states.shape
    E = representations.shape[0]
    C = 256
    n = H // C

    emb = jnp.take(representations, current_indices, axis=0)
    onehot = (jax.lax.iota(jnp.int32, E)[None, :]
              == jnp.asarray(current_expert_idx, jnp.int32)).astype(jnp.float32)
    b2 = b.reshape(1, H)

    out = pl.pallas_call(
        functools.partial(_fused, s_len=S, n_chunks=n),
        grid=(n,),
        in_specs=[
            pl.BlockSpec((B, S, C), lambda k: (0, 0, k)),
            pl.BlockSpec((H, C), lambda k: (0, k)),
            pl.BlockSpec((E, H), lambda k: (0, 0)),
            pl.BlockSpec((1, H), lambda k: (0, 0)),
            pl.BlockSpec((1, E), lambda k: (0, 0)),
        ],
        out_specs=pl.BlockSpec((1, 1), lambda k: (0, 0)),
        out_shape=jax.ShapeDtypeStruct((1, 1), jnp.float32),
        scratch_shapes=[pltpu.VMEM((B, E), jnp.float32)],
    )(hidden_states, W, emb, b2, onehot)
    return out[0, 0]
